# baseline (speedup 1.0000x reference)
import functools

import numpy as _np

import jax
import jax.numpy as jnp
from jax import lax
from jax.experimental import pallas as pl
from jax.experimental.pallas import tpu as pltpu
from jax.experimental.pallas import tpu_sc as plsc

N = 10000
NPAD = 10240
E = 320000
EPAD = E + 24
ROWB = 512

SC_NC = 2
SC_NW = 32
BLKN = 16
NBLK = NPAD // BLKN
BPT = NBLK // SC_NW
KCH = 16


def _mm_body(x_ref, w_ref, b_ref, o_ref):
    o_ref[...] = (
        jnp.dot(x_ref[...], w_ref[...], preferred_element_type=jnp.float32)
        + b_ref[...]
    )


def _matmul_bias(x, w, b):
    k = x.shape[1]
    m = w.shape[1]
    grid = NPAD // ROWB
    return pl.pallas_call(
        _mm_body,
        grid=(grid,),
        in_specs=[
            pl.BlockSpec((ROWB, k), lambda i: (i, 0)),
            pl.BlockSpec((k, m), lambda i: (0, 0)),
            pl.BlockSpec((1, m), lambda i: (0, 0)),
        ],
        out_specs=pl.BlockSpec((ROWB, m), lambda i: (i, 0)),
        out_shape=jax.ShapeDtypeStruct((NPAD, m), jnp.float32),
    )(x, w, b.reshape(1, m))


def _finish_body(heads, act, agg_ref, rinv_ref, wf_ref, b_ref, o_ref):
    d = agg_ref.shape[1] // heads
    acc = jnp.zeros_like(o_ref)
    for h in range(heads):
        blk = agg_ref[:, h * d:(h + 1) * d] * rinv_ref[:, h:h + 1]
        acc += jnp.dot(blk, wf_ref[h * d:(h + 1) * d, :],
                       preferred_element_type=jnp.float32)
    acc += b_ref[...]
    if act == "elu":
        acc = jnp.where(acc > 0, acc, jnp.exp(jnp.minimum(acc, 0.0)) - 1.0)
    o_ref[...] = acc


def _finish_layer(agg, rinv, wf, bf, heads, act):
    hd = agg.shape[1]
    m = wf.shape[1]
    grid = NPAD // ROWB
    return pl.pallas_call(
        functools.partial(_finish_body, heads, act),
        grid=(grid,),
        in_specs=[
            pl.BlockSpec((ROWB, hd), lambda i: (i, 0)),
            pl.BlockSpec((ROWB, 128), lambda i: (i, 0)),
            pl.BlockSpec((hd, m), lambda i: (0, 0)),
            pl.BlockSpec((1, m), lambda i: (0, 0)),
        ],
        out_specs=pl.BlockSpec((ROWB, m), lambda i: (i, 0)),
        out_shape=jax.ShapeDtypeStruct((NPAD, m), jnp.float32),
    )(agg, rinv, wf, bf.reshape(1, m))


def _sc_edge_body(heads, dim,
                  hs_hbm, hd_hbm, srcs_hbm, dsts_hbm, bst_hbm, attn_hbm,
                  agg_out, den_out,
                  agg_v, hd_v, rows_v, attn_v, den_v, sidx_v,
                  bst_v, didx_v, sem):
    hdf = heads * dim
    nj = dim // 16
    wid = lax.axis_index("s") * SC_NC + lax.axis_index("c")

    pltpu.sync_copy(bst_hbm, bst_v)
    pltpu.sync_copy(attn_hbm, attn_v)
    lanes = lax.iota(jnp.int32, 16)
    zero16 = jnp.zeros((16,), jnp.float32)
    onehots = [(1 - jnp.minimum(jnp.abs(lanes - h), 1)).astype(jnp.float32)
               for h in range(heads)]
    _gd = lax.GatherDimensionNumbers(
        offset_dims=(), collapsed_slice_dims=(0,), start_index_map=(0,))

    def lane_shuffle(v, idx):
        return lax.gather(v, idx[:, None], _gd, slice_sizes=(1,),
                          mode=lax.GatherScatterMode.PROMISE_IN_BOUNDS)

    def block_body(t, carry):
        b = t * SC_NW + wid
        nb0 = b * BLKN
        sev = plsc.load_gather(bst_v, [b + jnp.minimum(lanes, 1)])
        s = sev[0]
        e = sev[1]
        pltpu.sync_copy(hd_hbm.at[pl.ds(nb0, BLKN)], hd_v)

        def zero_body(r, carry2):
            def zrow(j, carry3):
                agg_v[r, pl.ds(j * 16, 16)] = zero16
                return carry3
            lax.fori_loop(0, hdf // 16, zrow, 0, unroll=8)
            den_v[r] = zero16
            return carry2
        lax.fori_loop(0, BLKN, zero_body, 0)

        s0 = (s // 8) * 8
        nch = (e - s0 + KCH - 1) // KCH

        def chunk_body(c, carry2):
            base = s0 + c * KCH
            pltpu.sync_copy(srcs_hbm.at[pl.ds(base, KCH)], sidx_v)
            pltpu.sync_copy(dsts_hbm.at[pl.ds(base, KCH)], didx_v)
            pltpu.async_copy(hs_hbm.at[sidx_v], rows_v, sem).wait()

            dv = plsc.load_gather(didx_v, [lanes])
            dvc = jnp.clip(dv - nb0, 0, BLKN - 1)
            dls = [dvc[k] for k in range(KCH)]
            vfs = []
            for k in range(KCH):
                pos = base + k
                valid = jnp.logical_and(pos >= s, pos < e)
                vfs.append(jnp.where(valid, 1.0, 0.0))

            for h in range(heads):
                def lg_body(j, accs):
                    off = h * dim + j * 16
                    av = attn_v[pl.ds(off, 16)]
                    out = []
                    for k in range(KCH):
                        u = (rows_v[k, pl.ds(off, 16)]
                             + hd_v[dls[k], pl.ds(off, 16)])
                        lr = jnp.maximum(u, 0.2 * u)
                        out.append(accs[k] + lr * av)
                    return tuple(out)
                accs = lax.fori_loop(0, nj, lg_body, (zero16,) * KCH)

                wvs = []
                for k in range(KCH):
                    acc = accs[k]
                    for sh in (8, 4, 2, 1):
                        acc = acc + lane_shuffle(
                            acc, jnp.bitwise_xor(lanes, sh))
                    wv = jnp.exp(acc) * jnp.broadcast_to(vfs[k], (16,))
                    plsc.addupdate(den_v.at[dls[k]], wv * onehots[h])
                    wvs.append(wv)

                def ag_body(j, carry3):
                    off = h * dim + j * 16
                    for k in range(KCH):
                        plsc.addupdate(agg_v.at[dls[k], pl.ds(off, 16)],
                                       wvs[k] * rows_v[k, pl.ds(off, 16)])
                    return carry3
                lax.fori_loop(0, nj, ag_body, 0)
            return carry2
        lax.fori_loop(0, nch, chunk_body, 0)

        def rv_body(r, carry2):
            den_v[r] = 1.0 / jnp.maximum(den_v[r], 1e-9)
            return carry2
        lax.fori_loop(0, BLKN, rv_body, 0)

        pltpu.sync_copy(agg_v, agg_out.at[pl.ds(nb0, BLKN)])
        pltpu.sync_copy(den_v, den_out.at[pl.ds(nb0, BLKN)])
        return carry
    lax.fori_loop(0, BPT, block_body, 0)


def _sc_edge_phase(hs, hd, srcs, dsts, bstarts, attn, heads, dim):
    hdf = heads * dim
    mesh = plsc.VectorSubcoreMesh(core_axis_name="c", subcore_axis_name="s")
    run = pl.kernel(
        functools.partial(_sc_edge_body, heads, dim),
        out_type=[jax.ShapeDtypeStruct((NPAD, hdf), jnp.float32),
                  jax.ShapeDtypeStruct((NPAD, 16), jnp.float32)],
        mesh=mesh,
        scratch_types=[
            pltpu.VMEM((BLKN, hdf), jnp.float32),
            pltpu.VMEM((BLKN, hdf), jnp.float32),
            pltpu.VMEM((KCH, hdf), jnp.float32),
            pltpu.VMEM((hdf,), jnp.float32),
            pltpu.VMEM((BLKN, 16), jnp.float32),
            pltpu.VMEM((KCH,), jnp.int32),
            pltpu.VMEM((648,), jnp.int32),
            pltpu.VMEM((KCH,), jnp.int32),
            pltpu.SemaphoreType.DMA,
        ],
        compiler_params=pltpu.CompilerParams(needs_layout_passes=False),
    )
    return run(hs, hd, srcs, dsts, bstarts, attn)


def _edge_prep(src, dst):
    dst_s, src_s = lax.sort((dst, src), num_keys=1)
    bstarts = jnp.searchsorted(
        dst_s, jnp.arange(0, NPAD + BLKN, BLKN, dtype=jnp.int32)
    ).astype(jnp.int32)
    bstarts = jnp.pad(bstarts, (0, 648 - (NBLK + 1)))
    src_p = jnp.pad(src_s, (0, EPAD - E))
    dst_p = jnp.pad(dst_s, (0, EPAD - E), constant_values=NPAD)
    return src_p, dst_p, bstarts


def _edge_phase(hs, hd, src, dst, attn, heads, dim):
    hs3 = hs[:N].reshape(N, heads, dim)
    hd3 = hd[:N].reshape(N, heads, dim)
    e = hs3[src] + hd3[dst]
    e = jnp.maximum(e, 0.2 * e)
    logits = (e * attn[None, :, :]).sum(-1)
    m = jnp.max(logits)
    ex = jnp.exp(logits - m)
    denom = jax.ops.segment_sum(ex, dst, num_segments=N)
    msg = ex[:, :, None] * hs3[src]
    agg = jax.ops.segment_sum(msg, dst, num_segments=N)
    rinv = 1.0 / jnp.maximum(denom, 1e-9)
    agg = jnp.pad(agg.reshape(N, heads * dim), ((0, NPAD - N), (0, 0)))
    rinv = jnp.pad(rinv, ((0, NPAD - N), (0, 128 - heads)))
    return agg, rinv


def kernel(feat, edge_index1, edge_index2, W1s, b1s, W1d, b1d, a1, Wf1, bf1,
           W2s, b2s, W2d, b2d, a2, Wf2, bf2):
    src1 = edge_index1[0].astype(jnp.int32)
    dst1 = edge_index1[1].astype(jnp.int32)
    src2 = edge_index2[0].astype(jnp.int32)
    dst2 = edge_index2[1].astype(jnp.int32)

    featp = jnp.pad(feat, ((0, NPAD - N), (0, 0)))

    w1 = jnp.concatenate([W1s, W1d], axis=1)
    b1 = jnp.concatenate([b1s, b1d])
    h1 = _matmul_bias(featp, w1, b1)
    hs1, hd1 = h1[:, :2048], h1[:, 2048:]

    src1p, dst1p, bst1 = _edge_prep(src1, dst1)
    agg1, rinv1 = _sc_edge_phase(hs1, hd1, src1p, dst1p, bst1,
                                 a1.reshape(-1), 4, 512)
    rinv1 = jnp.pad(rinv1, ((0, 0), (0, 112)))

    x1 = _finish_layer(agg1, rinv1, Wf1, bf1, 4, "elu")
    w2 = jnp.concatenate([W2s, W2d], axis=1)
    b2 = jnp.concatenate([b2s, b2d])
    h2 = _matmul_bias(x1, w2, b2)
    hs2, hd2 = h2[:, :256], h2[:, 256:]

    src2p, dst2p, bst2 = _edge_prep(src2, dst2)
    agg2, rinv2 = _sc_edge_phase(hs2, hd2, src2p, dst2p, bst2,
                                 a2.reshape(-1), 1, 256)
    rinv2 = jnp.pad(rinv2, ((0, 0), (0, 112)))

    z = _finish_layer(agg2, rinv2, Wf2, bf2, 1, "none")
    return z[:N]


# speedup vs baseline: 1.5993x; 1.5993x over previous
import functools

import numpy as _np

import jax
import jax.numpy as jnp
from jax import lax
from jax.experimental import pallas as pl
from jax.experimental.pallas import tpu as pltpu
from jax.experimental.pallas import tpu_sc as plsc

N = 10000
NPAD = 10240
E = 320000
EPAD = E + 24
ROWB = 512

SC_NC = 2
SC_NW = 32
BLKN = 16
NBLK = NPAD // BLKN
BPT = NBLK // SC_NW
KCH = 16


def _mm_body(x_ref, w_ref, b_ref, o_ref):
    o_ref[...] = (
        jnp.dot(x_ref[...], w_ref[...], preferred_element_type=jnp.float32)
        + b_ref[...]
    )


def _matmul_bias(x, w, b):
    k = x.shape[1]
    m = w.shape[1]
    grid = NPAD // ROWB
    return pl.pallas_call(
        _mm_body,
        grid=(grid,),
        in_specs=[
            pl.BlockSpec((ROWB, k), lambda i: (i, 0)),
            pl.BlockSpec((k, m), lambda i: (0, 0)),
            pl.BlockSpec((1, m), lambda i: (0, 0)),
        ],
        out_specs=pl.BlockSpec((ROWB, m), lambda i: (i, 0)),
        out_shape=jax.ShapeDtypeStruct((NPAD, m), jnp.float32),
    )(x, w, b.reshape(1, m))


def _finish_body(heads, act, agg_ref, rinv_ref, wf_ref, b_ref, o_ref):
    d = agg_ref.shape[1] // heads
    acc = jnp.zeros_like(o_ref)
    for h in range(heads):
        blk = agg_ref[:, h * d:(h + 1) * d] * rinv_ref[:, h:h + 1]
        acc += jnp.dot(blk, wf_ref[h * d:(h + 1) * d, :],
                       preferred_element_type=jnp.float32)
    acc += b_ref[...]
    if act == "elu":
        acc = jnp.where(acc > 0, acc, jnp.exp(jnp.minimum(acc, 0.0)) - 1.0)
    o_ref[...] = acc


def _finish_layer(agg, rinv, wf, bf, heads, act):
    hd = agg.shape[1]
    m = wf.shape[1]
    grid = NPAD // ROWB
    return pl.pallas_call(
        functools.partial(_finish_body, heads, act),
        grid=(grid,),
        in_specs=[
            pl.BlockSpec((ROWB, hd), lambda i: (i, 0)),
            pl.BlockSpec((ROWB, 128), lambda i: (i, 0)),
            pl.BlockSpec((hd, m), lambda i: (0, 0)),
            pl.BlockSpec((1, m), lambda i: (0, 0)),
        ],
        out_specs=pl.BlockSpec((ROWB, m), lambda i: (i, 0)),
        out_shape=jax.ShapeDtypeStruct((NPAD, m), jnp.float32),
    )(agg, rinv, wf, bf.reshape(1, m))


def _sc_edge_body(heads, dim,
                  hs_hbm, hd_hbm, srcs_hbm, dsts_hbm, bst_hbm, attn_hbm,
                  agg_out, den_out,
                  agg_v, hd_v, rows_v, attn_v, den_v, sidx_v,
                  bst_v, didx_v, sem):
    hdf = heads * dim
    nj = dim // 16
    wid = lax.axis_index("s") * SC_NC + lax.axis_index("c")

    pltpu.sync_copy(bst_hbm, bst_v)
    pltpu.sync_copy(attn_hbm, attn_v)
    lanes = lax.iota(jnp.int32, 16)
    zero16 = jnp.zeros((16,), jnp.float32)
    onehots = [(1 - jnp.minimum(jnp.abs(lanes - h), 1)).astype(jnp.float32)
               for h in range(heads)]
    _gd = lax.GatherDimensionNumbers(
        offset_dims=(), collapsed_slice_dims=(0,), start_index_map=(0,))

    def lane_shuffle(v, idx):
        return lax.gather(v, idx[:, None], _gd, slice_sizes=(1,),
                          mode=lax.GatherScatterMode.PROMISE_IN_BOUNDS)

    def block_body(t, carry):
        b = t * SC_NW + wid
        nb0 = b * BLKN
        sev = plsc.load_gather(bst_v, [b + jnp.minimum(lanes, 1)])
        s = sev[0]
        e = sev[1]
        pltpu.sync_copy(hd_hbm.at[pl.ds(nb0, BLKN)], hd_v)

        def zero_body(r, carry2):
            def zrow(j, carry3):
                agg_v[r, pl.ds(j * 16, 16)] = zero16
                return carry3
            lax.fori_loop(0, hdf // 16, zrow, 0, unroll=8)
            den_v[r] = zero16
            return carry2
        lax.fori_loop(0, BLKN, zero_body, 0)

        s0 = (s // 8) * 8
        nch = (e - s0 + KCH - 1) // KCH

        def chunk_body(c, carry2):
            base = s0 + c * KCH
            pltpu.sync_copy(srcs_hbm.at[pl.ds(base, KCH)], sidx_v)
            pltpu.sync_copy(dsts_hbm.at[pl.ds(base, KCH)], didx_v)
            pltpu.async_copy(hs_hbm.at[sidx_v], rows_v, sem).wait()

            dv = plsc.load_gather(didx_v, [lanes])
            dvc = jnp.clip(dv - nb0, 0, BLKN - 1)
            dls = [dvc[k] for k in range(KCH)]
            vfs = []
            for k in range(KCH):
                pos = base + k
                valid = jnp.logical_and(pos >= s, pos < e)
                vfs.append(jnp.where(valid, 1.0, 0.0))

            for h in range(heads):
                def lg_body(j, accs):
                    off = h * dim + j * 16
                    av = attn_v[pl.ds(off, 16)]
                    out = []
                    for k in range(KCH):
                        u = (rows_v[k, pl.ds(off, 16)]
                             + hd_v[dls[k], pl.ds(off, 16)])
                        lr = jnp.maximum(u, 0.2 * u)
                        out.append(accs[k] + lr * av)
                    return tuple(out)
                accs = lax.fori_loop(0, nj, lg_body, (zero16,) * KCH)

                wvs = []
                for k in range(KCH):
                    acc = accs[k]
                    for sh in (8, 4, 2, 1):
                        acc = acc + lane_shuffle(
                            acc, jnp.bitwise_xor(lanes, sh))
                    wv = jnp.exp(acc) * jnp.broadcast_to(vfs[k], (16,))
                    plsc.addupdate(den_v.at[dls[k]], wv * onehots[h])
                    wvs.append(wv)

                def ag_body(j, carry3):
                    off = h * dim + j * 16
                    rs = [rows_v[k, pl.ds(off, 16)] for k in range(KCH)]
                    ms = [wvs[k] * rs[k] for k in range(KCH)]
                    for k in range(KCH):
                        plsc.addupdate(agg_v.at[dls[k], pl.ds(off, 16)],
                                       ms[k])
                    return carry3
                lax.fori_loop(0, nj, ag_body, 0)
            return carry2
        lax.fori_loop(0, nch, chunk_body, 0)

        def rv_body(r, carry2):
            den_v[r] = 1.0 / jnp.maximum(den_v[r], 1e-9)
            return carry2
        lax.fori_loop(0, BLKN, rv_body, 0)

        pltpu.sync_copy(agg_v, agg_out.at[pl.ds(nb0, BLKN)])
        pltpu.sync_copy(den_v, den_out.at[pl.ds(nb0, BLKN)])
        return carry
    lax.fori_loop(0, BPT, block_body, 0)


def _sc_edge_phase(hs, hd, srcs, dsts, bstarts, attn, heads, dim):
    hdf = heads * dim
    mesh = plsc.VectorSubcoreMesh(core_axis_name="c", subcore_axis_name="s")
    run = pl.kernel(
        functools.partial(_sc_edge_body, heads, dim),
        out_type=[jax.ShapeDtypeStruct((NPAD, hdf), jnp.float32),
                  jax.ShapeDtypeStruct((NPAD, 16), jnp.float32)],
        mesh=mesh,
        scratch_types=[
            pltpu.VMEM((BLKN, hdf), jnp.float32),
            pltpu.VMEM((BLKN, hdf), jnp.float32),
            pltpu.VMEM((KCH, hdf), jnp.float32),
            pltpu.VMEM((hdf,), jnp.float32),
            pltpu.VMEM((BLKN, 16), jnp.float32),
            pltpu.VMEM((KCH,), jnp.int32),
            pltpu.VMEM((648,), jnp.int32),
            pltpu.VMEM((KCH,), jnp.int32),
            pltpu.SemaphoreType.DMA,
        ],
        compiler_params=pltpu.CompilerParams(needs_layout_passes=False),
    )
    return run(hs, hd, srcs, dsts, bstarts, attn)


def _edge_prep(src, dst):
    dst_s, src_s = lax.sort((dst, src), num_keys=1)
    bstarts = jnp.searchsorted(
        dst_s, jnp.arange(0, NPAD + BLKN, BLKN, dtype=jnp.int32)
    ).astype(jnp.int32)
    bstarts = jnp.pad(bstarts, (0, 648 - (NBLK + 1)))
    src_p = jnp.pad(src_s, (0, EPAD - E))
    dst_p = jnp.pad(dst_s, (0, EPAD - E), constant_values=NPAD)
    return src_p, dst_p, bstarts


def _edge_phase(hs, hd, src, dst, attn, heads, dim):
    hs3 = hs[:N].reshape(N, heads, dim)
    hd3 = hd[:N].reshape(N, heads, dim)
    e = hs3[src] + hd3[dst]
    e = jnp.maximum(e, 0.2 * e)
    logits = (e * attn[None, :, :]).sum(-1)
    m = jnp.max(logits)
    ex = jnp.exp(logits - m)
    denom = jax.ops.segment_sum(ex, dst, num_segments=N)
    msg = ex[:, :, None] * hs3[src]
    agg = jax.ops.segment_sum(msg, dst, num_segments=N)
    rinv = 1.0 / jnp.maximum(denom, 1e-9)
    agg = jnp.pad(agg.reshape(N, heads * dim), ((0, NPAD - N), (0, 0)))
    rinv = jnp.pad(rinv, ((0, NPAD - N), (0, 128 - heads)))
    return agg, rinv


def kernel(feat, edge_index1, edge_index2, W1s, b1s, W1d, b1d, a1, Wf1, bf1,
           W2s, b2s, W2d, b2d, a2, Wf2, bf2):
    src1 = edge_index1[0].astype(jnp.int32)
    dst1 = edge_index1[1].astype(jnp.int32)
    src2 = edge_index2[0].astype(jnp.int32)
    dst2 = edge_index2[1].astype(jnp.int32)

    featp = jnp.pad(feat, ((0, NPAD - N), (0, 0)))

    w1 = jnp.concatenate([W1s, W1d], axis=1)
    b1 = jnp.concatenate([b1s, b1d])
    h1 = _matmul_bias(featp, w1, b1)
    hs1, hd1 = h1[:, :2048], h1[:, 2048:]

    src1p, dst1p, bst1 = _edge_prep(src1, dst1)
    agg1, rinv1 = _sc_edge_phase(hs1, hd1, src1p, dst1p, bst1,
                                 a1.reshape(-1), 4, 512)
    rinv1 = jnp.pad(rinv1, ((0, 0), (0, 112)))

    x1 = _finish_layer(agg1, rinv1, Wf1, bf1, 4, "elu")
    w2 = jnp.concatenate([W2s, W2d], axis=1)
    b2 = jnp.concatenate([b2s, b2d])
    h2 = _matmul_bias(x1, w2, b2)
    hs2, hd2 = h2[:, :256], h2[:, 256:]

    src2p, dst2p, bst2 = _edge_prep(src2, dst2)
    agg2, rinv2 = _sc_edge_phase(hs2, hd2, src2p, dst2p, bst2,
                                 a2.reshape(-1), 1, 256)
    rinv2 = jnp.pad(rinv2, ((0, 0), (0, 112)))

    z = _finish_layer(agg2, rinv2, Wf2, bf2, 1, "none")
    return z[:N]


# speedup vs baseline: 1.5995x; 1.0001x over previous
import functools

import jax
import jax.numpy as jnp
from jax import lax
from jax.experimental import pallas as pl
from jax.experimental.pallas import tpu as pltpu
from jax.experimental.pallas import tpu_sc as plsc

N = 10000
NPAD = 10240
E = 320000
EPAD = E + 24
ROWB = 512

SC_NC = 2
SC_NW = 32
BLKN = 16
NBLK = NPAD // BLKN
BPT = NBLK // SC_NW
KCH = 16


def _mm_body(x_ref, w_ref, b_ref, o_ref):
    o_ref[...] = (
        jnp.dot(x_ref[...], w_ref[...], preferred_element_type=jnp.float32)
        + b_ref[...]
    )


def _matmul_bias(x, w, b):
    k = x.shape[1]
    m = w.shape[1]
    grid = NPAD // ROWB
    return pl.pallas_call(
        _mm_body,
        grid=(grid,),
        in_specs=[
            pl.BlockSpec((ROWB, k), lambda i: (i, 0)),
            pl.BlockSpec((k, m), lambda i: (0, 0)),
            pl.BlockSpec((1, m), lambda i: (0, 0)),
        ],
        out_specs=pl.BlockSpec((ROWB, m), lambda i: (i, 0)),
        out_shape=jax.ShapeDtypeStruct((NPAD, m), jnp.float32),
    )(x, w, b.reshape(1, m))


def _finish_body(heads, act, agg_ref, rinv_ref, wf_ref, b_ref, o_ref):
    d = agg_ref.shape[1] // heads
    acc = jnp.zeros_like(o_ref)
    for h in range(heads):
        blk = agg_ref[:, h * d:(h + 1) * d] * rinv_ref[:, h:h + 1]
        acc += jnp.dot(blk, wf_ref[h * d:(h + 1) * d, :],
                       preferred_element_type=jnp.float32)
    acc += b_ref[...]
    if act == "elu":
        acc = jnp.where(acc > 0, acc, jnp.exp(jnp.minimum(acc, 0.0)) - 1.0)
    o_ref[...] = acc


def _finish_layer(agg, rinv, wf, bf, heads, act):
    hd = agg.shape[1]
    m = wf.shape[1]
    grid = NPAD // ROWB
    return pl.pallas_call(
        functools.partial(_finish_body, heads, act),
        grid=(grid,),
        in_specs=[
            pl.BlockSpec((ROWB, hd), lambda i: (i, 0)),
            pl.BlockSpec((ROWB, 128), lambda i: (i, 0)),
            pl.BlockSpec((hd, m), lambda i: (0, 0)),
            pl.BlockSpec((1, m), lambda i: (0, 0)),
        ],
        out_specs=pl.BlockSpec((ROWB, m), lambda i: (i, 0)),
        out_shape=jax.ShapeDtypeStruct((NPAD, m), jnp.float32),
    )(agg, rinv, wf, bf.reshape(1, m))


def _sc_edge_body(heads, dim,
                  hs_hbm, hd_hbm, srcs_hbm, dsts_hbm, bst_hbm, attn_hbm,
                  agg_out, den_out,
                  agg_v, hd_v, rows_v, attn_v, den_v, sidx_v,
                  bst_v, didx_v, sem):
    hdf = heads * dim
    nj = dim // 16
    wid = lax.axis_index("s") * SC_NC + lax.axis_index("c")

    pltpu.sync_copy(bst_hbm, bst_v)
    pltpu.sync_copy(attn_hbm, attn_v)
    lanes = lax.iota(jnp.int32, 16)
    zero16 = jnp.zeros((16,), jnp.float32)
    onehots = [(1 - jnp.minimum(jnp.abs(lanes - h), 1)).astype(jnp.float32)
               for h in range(heads)]
    _gd = lax.GatherDimensionNumbers(
        offset_dims=(), collapsed_slice_dims=(0,), start_index_map=(0,))

    def lane_shuffle(v, idx):
        return lax.gather(v, idx[:, None], _gd, slice_sizes=(1,),
                          mode=lax.GatherScatterMode.PROMISE_IN_BOUNDS)

    def block_body(t, carry):
        b = t * SC_NW + wid
        nb0 = b * BLKN
        sev = plsc.load_gather(bst_v, [b + jnp.minimum(lanes, 1)])
        s = sev[0]
        e = sev[1]
        pltpu.sync_copy(hd_hbm.at[pl.ds(nb0, BLKN)], hd_v)

        def zero_body(r, carry2):
            def zrow(j, carry3):
                agg_v[r, pl.ds(j * 16, 16)] = zero16
                return carry3
            lax.fori_loop(0, hdf // 16, zrow, 0, unroll=8)
            den_v[r] = zero16
            return carry2
        lax.fori_loop(0, BLKN, zero_body, 0)

        s0 = (s // 8) * 8
        nch = (e - s0 + KCH - 1) // KCH

        def chunk_body(c, carry2):
            base = s0 + c * KCH
            pltpu.sync_copy(srcs_hbm.at[pl.ds(base, KCH)], sidx_v)
            pltpu.sync_copy(dsts_hbm.at[pl.ds(base, KCH)], didx_v)
            pltpu.async_copy(hs_hbm.at[sidx_v], rows_v, sem).wait()

            dv = plsc.load_gather(didx_v, [lanes])
            dvc = jnp.clip(dv - nb0, 0, BLKN - 1)
            dls = [dvc[k] for k in range(KCH)]
            vfs = []
            for k in range(KCH):
                pos = base + k
                valid = jnp.logical_and(pos >= s, pos < e)
                vfs.append(jnp.where(valid, 1.0, 0.0))

            for h in range(heads):
                def lg_body(j, accs):
                    off = h * dim + j * 16
                    av = attn_v[pl.ds(off, 16)]
                    out = []
                    for k in range(KCH):
                        u = (rows_v[k, pl.ds(off, 16)]
                             + hd_v[dls[k], pl.ds(off, 16)])
                        lr = jnp.maximum(u, 0.2 * u)
                        out.append(accs[k] + lr * av)
                    return tuple(out)
                accs = lax.fori_loop(0, nj, lg_body, (zero16,) * KCH)

                wvs = []
                for k in range(KCH):
                    acc = accs[k]
                    for sh in (8, 4, 2, 1):
                        acc = acc + lane_shuffle(
                            acc, jnp.bitwise_xor(lanes, sh))
                    wv = jnp.exp(acc) * jnp.broadcast_to(vfs[k], (16,))
                    plsc.addupdate(den_v.at[dls[k]], wv * onehots[h])
                    wvs.append(wv)

                def ag_body(j, carry3):
                    off = h * dim + j * 16
                    rs = [rows_v[k, pl.ds(off, 16)] for k in range(KCH)]
                    ms = [wvs[k] * rs[k] for k in range(KCH)]
                    for k in range(KCH):
                        plsc.addupdate(agg_v.at[dls[k], pl.ds(off, 16)],
                                       ms[k])
                    return carry3
                lax.fori_loop(0, nj, ag_body, 0)
            return carry2
        lax.fori_loop(0, nch, chunk_body, 0)

        def rv_body(r, carry2):
            den_v[r] = 1.0 / jnp.maximum(den_v[r], 1e-9)
            return carry2
        lax.fori_loop(0, BLKN, rv_body, 0)

        pltpu.sync_copy(agg_v, agg_out.at[pl.ds(nb0, BLKN)])
        pltpu.sync_copy(den_v, den_out.at[pl.ds(nb0, BLKN)])
        return carry
    lax.fori_loop(0, BPT, block_body, 0)


def _sc_edge_phase(hs, hd, srcs, dsts, bstarts, attn, heads, dim):
    hdf = heads * dim
    mesh = plsc.VectorSubcoreMesh(core_axis_name="c", subcore_axis_name="s")
    run = pl.kernel(
        functools.partial(_sc_edge_body, heads, dim),
        out_type=[jax.ShapeDtypeStruct((NPAD, hdf), jnp.float32),
                  jax.ShapeDtypeStruct((NPAD, 16), jnp.float32)],
        mesh=mesh,
        scratch_types=[
            pltpu.VMEM((BLKN, hdf), jnp.float32),
            pltpu.VMEM((BLKN, hdf), jnp.float32),
            pltpu.VMEM((KCH, hdf), jnp.float32),
            pltpu.VMEM((hdf,), jnp.float32),
            pltpu.VMEM((BLKN, 16), jnp.float32),
            pltpu.VMEM((KCH,), jnp.int32),
            pltpu.VMEM((648,), jnp.int32),
            pltpu.VMEM((KCH,), jnp.int32),
            pltpu.SemaphoreType.DMA,
        ],
        compiler_params=pltpu.CompilerParams(needs_layout_passes=False),
    )
    return run(hs, hd, srcs, dsts, bstarts, attn)


def _edge_prep(src, dst):
    dst_s, src_s = lax.sort((dst, src), num_keys=1)
    bstarts = jnp.searchsorted(
        dst_s, jnp.arange(0, NPAD + BLKN, BLKN, dtype=jnp.int32)
    ).astype(jnp.int32)
    bstarts = jnp.pad(bstarts, (0, 648 - (NBLK + 1)))
    src_p = jnp.pad(src_s, (0, EPAD - E))
    dst_p = jnp.pad(dst_s, (0, EPAD - E), constant_values=NPAD)
    return src_p, dst_p, bstarts


def kernel(feat, edge_index1, edge_index2, W1s, b1s, W1d, b1d, a1, Wf1, bf1,
           W2s, b2s, W2d, b2d, a2, Wf2, bf2):
    src1 = edge_index1[0].astype(jnp.int32)
    dst1 = edge_index1[1].astype(jnp.int32)
    src2 = edge_index2[0].astype(jnp.int32)
    dst2 = edge_index2[1].astype(jnp.int32)

    featp = jnp.pad(feat, ((0, NPAD - N), (0, 0)))

    w1 = jnp.concatenate([W1s, W1d], axis=1)
    b1 = jnp.concatenate([b1s, b1d])
    h1 = _matmul_bias(featp, w1, b1)
    hs1, hd1 = h1[:, :2048], h1[:, 2048:]

    src1p, dst1p, bst1 = _edge_prep(src1, dst1)
    agg1, rinv1 = _sc_edge_phase(hs1, hd1, src1p, dst1p, bst1,
                                 a1.reshape(-1), 4, 512)
    rinv1 = jnp.pad(rinv1, ((0, 0), (0, 112)))

    x1 = _finish_layer(agg1, rinv1, Wf1, bf1, 4, "elu")
    w2 = jnp.concatenate([W2s, W2d], axis=1)
    b2 = jnp.concatenate([b2s, b2d])
    h2 = _matmul_bias(x1, w2, b2)
    hs2, hd2 = h2[:, :256], h2[:, 256:]

    src2p, dst2p, bst2 = _edge_prep(src2, dst2)
    agg2, rinv2 = _sc_edge_phase(hs2, hd2, src2p, dst2p, bst2,
                                 a2.reshape(-1), 1, 256)
    rinv2 = jnp.pad(rinv2, ((0, 0), (0, 112)))

    z = _finish_layer(agg2, rinv2, Wf2, bf2, 1, "none")
    return z[:N]
